# Initial kernel scaffold; baseline (speedup 1.0000x reference)
import jax
import jax.numpy as jnp
from jax.experimental import pallas as pl


def kernel(x, edge_index, Wl1, Wr1, att1, b1, Wl2, Wr2, att2, b2, Wl3, Wr3, att3, b3):
    raise NotImplementedError("write your pallas kernel here")


# speedup vs baseline: 7.3848x; 7.3848x over previous
import functools

import jax
import jax.numpy as jnp
from jax import lax
from jax.experimental import pallas as pl
from jax.experimental.pallas import tpu as pltpu
from jax.experimental.pallas import tpu_sc as plsc

N = 10000
NP = 10240
H = 128
ROWW = 144
C = 128
NW = 32
E_TOT = 320000 + N
K = -(-E_TOT // (NW * C))
EP = NW * C * K
ROWS_PER_TILE = NP // 16


def _tc_matmul2(h, wl, wr):
    kd = h.shape[1]

    def body(h_ref, wl_ref, wr_ref, xl_ref, xr_ref):
        hb = h_ref[...]
        xl_ref[...] = jnp.dot(hb, wl_ref[...], preferred_element_type=jnp.float32)
        xr_ref[...] = jnp.dot(hb, wr_ref[...], preferred_element_type=jnp.float32)

    return pl.pallas_call(
        body,
        grid=(NP // 512,),
        in_specs=[
            pl.BlockSpec((512, kd), lambda i: (i, 0)),
            pl.BlockSpec((kd, H), lambda i: (0, 0)),
            pl.BlockSpec((kd, H), lambda i: (0, 0)),
        ],
        out_specs=[
            pl.BlockSpec((512, H), lambda i: (i, 0)),
            pl.BlockSpec((512, H), lambda i: (i, 0)),
        ],
        out_shape=[
            jax.ShapeDtypeStruct((NP, H), jnp.float32),
            jax.ShapeDtypeStruct((NP, H), jnp.float32),
        ],
    )(h, wl, wr)


def _tc_finish_mid(ao, asum, b, wl, wr):
    def body(a0_ref, a1_ref, s0_ref, s1_ref, b_ref, wl_ref, wr_ref,
             xl_ref, xr_ref):
        o = a0_ref[0] + a1_ref[0]
        s = s0_ref[0][:, 0:1] + s1_ref[0][:, 0:1]
        h = o / (s + 1e-16) + b_ref[...]
        h = jnp.maximum(h, 0.0)
        xl_ref[...] = jnp.dot(h, wl_ref[...], preferred_element_type=jnp.float32)
        xr_ref[...] = jnp.dot(h, wr_ref[...], preferred_element_type=jnp.float32)

    return pl.pallas_call(
        body,
        grid=(NP // 512,),
        in_specs=[
            pl.BlockSpec((1, 512, H), lambda i: (0, i, 0)),
            pl.BlockSpec((1, 512, H), lambda i: (1, i, 0)),
            pl.BlockSpec((1, 512, 16), lambda i: (0, i, 0)),
            pl.BlockSpec((1, 512, 16), lambda i: (1, i, 0)),
            pl.BlockSpec((H,), lambda i: (0,)),
            pl.BlockSpec((H, H), lambda i: (0, 0)),
            pl.BlockSpec((H, H), lambda i: (0, 0)),
        ],
        out_specs=[
            pl.BlockSpec((512, H), lambda i: (i, 0)),
            pl.BlockSpec((512, H), lambda i: (i, 0)),
        ],
        out_shape=[
            jax.ShapeDtypeStruct((NP, H), jnp.float32),
            jax.ShapeDtypeStruct((NP, H), jnp.float32),
        ],
    )(ao, ao, asum, asum, b, wl, wr)


def _tc_finish_last(ao, asum, b):
    def body(a0_ref, a1_ref, s0_ref, s1_ref, b_ref, h_ref):
        o = a0_ref[0] + a1_ref[0]
        s = s0_ref[0][:, 0:1] + s1_ref[0][:, 0:1]
        h_ref[...] = o / (s + 1e-16) + b_ref[...]

    return pl.pallas_call(
        body,
        grid=(NP // 512,),
        in_specs=[
            pl.BlockSpec((1, 512, H), lambda i: (0, i, 0)),
            pl.BlockSpec((1, 512, H), lambda i: (1, i, 0)),
            pl.BlockSpec((1, 512, 16), lambda i: (0, i, 0)),
            pl.BlockSpec((1, 512, 16), lambda i: (1, i, 0)),
            pl.BlockSpec((H,), lambda i: (0,)),
        ],
        out_specs=pl.BlockSpec((512, H), lambda i: (i, 0)),
        out_shape=jax.ShapeDtypeStruct((NP, H), jnp.float32),
    )(ao, ao, asum, asum, b)


def _sc_edge_kernel(xl_hbm, src_hbm, dst_hbm, xr_hbm, att_hbm,
                    out_hbm, s_hbm,
                    out_shared, s_shared, att_v, idx_s, idx_d,
                    xl_rows, xr_rows, e_stage, sem_a, sem_b):
    cid = lax.axis_index("c")
    sid = lax.axis_index("s")

    pltpu.sync_copy(att_hbm, att_v)

    def zero_row(i, _):
        for k8 in range(8):
            xl_rows[i, pl.ds(16 * k8, 16)] = jnp.zeros((16,), jnp.float32)
        e_stage[i, :] = jnp.zeros((16,), jnp.float32)
        return 0

    lax.fori_loop(0, C, zero_row, 0)
    r0 = sid * ROWS_PER_TILE
    for j in range(ROWS_PER_TILE // C):
        pltpu.sync_copy(xl_rows, out_shared.at[pl.ds(r0 + j * C, C)])
        pltpu.sync_copy(e_stage, s_shared.at[pl.ds(r0 + j * C, C)])
    plsc.subcore_barrier()

    wid = cid * 16 + sid

    def chunk_body(k, _):
        base = (wid * K + k) * C
        pltpu.sync_copy(src_hbm.at[pl.ds(base, C)], idx_s)
        pltpu.sync_copy(dst_hbm.at[pl.ds(base, C)], idx_d)
        cp_a = pltpu.async_copy(xl_hbm.at[idx_s], xl_rows, sem_a)
        cp_b = pltpu.async_copy(xr_hbm.at[idx_d], xr_rows, sem_b)
        cp_a.wait()
        cp_b.wait()

        def edge_body(i, _):
            als = [xl_rows[i, pl.ds(16 * k8, 16)] for k8 in range(8)]
            p = jnp.zeros((16,), jnp.float32)
            for k8 in range(8):
                m = als[k8] + xr_rows[i, pl.ds(16 * k8, 16)]
                m = jnp.maximum(m, 0.2 * m)
                p = p + m * att_v[pl.ds(16 * k8, 16)]
            logit = jnp.sum(p)
            ev = jnp.exp(lax.broadcast(logit, (16,)))
            for k8 in range(8):
                xl_rows[i, pl.ds(16 * k8, 16)] = als[k8] * ev
            e_stage[i, :] = ev
            return 0

        lax.fori_loop(0, C, edge_body, 0)
        pltpu.sync_copy(xl_rows, out_shared.at[idx_d], add=True)
        pltpu.sync_copy(e_stage, s_shared.at[idx_d], add=True)
        return 0

    lax.fori_loop(0, K, chunk_body, 0)
    plsc.subcore_barrier()
    pltpu.sync_copy(out_shared.at[pl.ds(r0, ROWS_PER_TILE)],
                    out_hbm.at[cid, pl.ds(r0, ROWS_PER_TILE)])
    pltpu.sync_copy(s_shared.at[pl.ds(r0, ROWS_PER_TILE)],
                    s_hbm.at[cid, pl.ds(r0, ROWS_PER_TILE)])


def _sc_edge(xl, xr, att, src, dst):
    mesh = plsc.VectorSubcoreMesh(core_axis_name="c", subcore_axis_name="s")
    f = pl.kernel(
        _sc_edge_kernel,
        out_type=[
            jax.ShapeDtypeStruct((2, NP, H), jnp.float32),
            jax.ShapeDtypeStruct((2, NP, 16), jnp.float32),
        ],
        mesh=mesh,
        scratch_types=[
            pltpu.VMEM_SHARED((NP, H), jnp.float32),
            pltpu.VMEM_SHARED((NP, 16), jnp.float32),
            pltpu.VMEM((H,), jnp.float32),
            pltpu.VMEM((C,), jnp.int32),
            pltpu.VMEM((C,), jnp.int32),
            pltpu.VMEM((C, H), jnp.float32),
            pltpu.VMEM((C, H), jnp.float32),
            pltpu.VMEM((C, 16), jnp.float32),
            pltpu.SemaphoreType.DMA,
            pltpu.SemaphoreType.DMA,
        ],
        compiler_params=pltpu.CompilerParams(needs_layout_passes=False,
                                             use_tc_tiling_on_sc=False),
    )
    return f(xl, src, dst, xr, att)


def kernel(x, edge_index, Wl1, Wr1, att1, b1, Wl2, Wr2, att2, b2,
           Wl3, Wr3, att3, b3):
    x = x.astype(jnp.float32)
    feat = jnp.zeros((NP, 8), jnp.float32).at[:N, :6].set(x[:, 4:10])
    wl1 = jnp.zeros((8, H), jnp.float32).at[:6].set(Wl1)
    wr1 = jnp.zeros((8, H), jnp.float32).at[:6].set(Wr1)

    loops = jnp.arange(N, dtype=jnp.int32)
    pad = EP - E_TOT
    src = jnp.concatenate([edge_index[0].astype(jnp.int32), loops,
                           jnp.zeros((pad,), jnp.int32)])
    dst = jnp.concatenate([edge_index[1].astype(jnp.int32), loops,
                           jnp.full((pad,), NP - 1, jnp.int32)])

    xl, xr = _tc_matmul2(feat, wl1, wr1)
    ao, asum = _sc_edge(xl, xr, att1, src, dst)
    xl, xr = _tc_finish_mid(ao, asum, b1, Wl2, Wr2)
    ao, asum = _sc_edge(xl, xr, att2, src, dst)
    xl, xr = _tc_finish_mid(ao, asum, b2, Wl3, Wr3)
    ao, asum = _sc_edge(xl, xr, att3, src, dst)
    h = _tc_finish_last(ao, asum, b3)
    return h[:N]


# speedup vs baseline: 12.3455x; 1.6717x over previous
import functools

import jax
import jax.numpy as jnp
from jax import lax
from jax.experimental import pallas as pl
from jax.experimental.pallas import tpu as pltpu
from jax.experimental.pallas import tpu_sc as plsc

N = 10000
NP = 10240
H = 128
C = 64
NW = 32
E_TOT = 320000 + N
K = 4 * (-(-E_TOT // (NW * C * 4)))
EP = NW * C * K
ROWS_PER_TILE = NP // 16


def _tc_matmul2(h, wl, wr):
    kd = h.shape[1]

    def body(h_ref, wl_ref, wr_ref, xl_ref, xr_ref):
        hb = h_ref[...]
        xl_ref[...] = jnp.dot(hb, wl_ref[...], preferred_element_type=jnp.float32)
        xr_ref[...] = jnp.dot(hb, wr_ref[...], preferred_element_type=jnp.float32)

    return pl.pallas_call(
        body,
        grid=(NP // 512,),
        in_specs=[
            pl.BlockSpec((512, kd), lambda i: (i, 0)),
            pl.BlockSpec((kd, H), lambda i: (0, 0)),
            pl.BlockSpec((kd, H), lambda i: (0, 0)),
        ],
        out_specs=[
            pl.BlockSpec((512, H), lambda i: (i, 0)),
            pl.BlockSpec((512, H), lambda i: (i, 0)),
        ],
        out_shape=[
            jax.ShapeDtypeStruct((NP, H), jnp.float32),
            jax.ShapeDtypeStruct((NP, H), jnp.float32),
        ],
    )(h, wl, wr)


def _tc_finish_mid(ao, asum, b, wl, wr):
    def body(a0_ref, a1_ref, s0_ref, s1_ref, b_ref, wl_ref, wr_ref,
             xl_ref, xr_ref):
        o = a0_ref[0] + a1_ref[0]
        s = s0_ref[0][:, 0:1] + s1_ref[0][:, 0:1]
        h = o / (s + 1e-16) + b_ref[...]
        h = jnp.maximum(h, 0.0)
        xl_ref[...] = jnp.dot(h, wl_ref[...], preferred_element_type=jnp.float32)
        xr_ref[...] = jnp.dot(h, wr_ref[...], preferred_element_type=jnp.float32)

    return pl.pallas_call(
        body,
        grid=(NP // 512,),
        in_specs=[
            pl.BlockSpec((1, 512, H), lambda i: (0, i, 0)),
            pl.BlockSpec((1, 512, H), lambda i: (1, i, 0)),
            pl.BlockSpec((1, 512, 16), lambda i: (0, i, 0)),
            pl.BlockSpec((1, 512, 16), lambda i: (1, i, 0)),
            pl.BlockSpec((H,), lambda i: (0,)),
            pl.BlockSpec((H, H), lambda i: (0, 0)),
            pl.BlockSpec((H, H), lambda i: (0, 0)),
        ],
        out_specs=[
            pl.BlockSpec((512, H), lambda i: (i, 0)),
            pl.BlockSpec((512, H), lambda i: (i, 0)),
        ],
        out_shape=[
            jax.ShapeDtypeStruct((NP, H), jnp.float32),
            jax.ShapeDtypeStruct((NP, H), jnp.float32),
        ],
    )(ao, ao, asum, asum, b, wl, wr)


def _tc_finish_last(ao, asum, b):
    def body(a0_ref, a1_ref, s0_ref, s1_ref, b_ref, h_ref):
        o = a0_ref[0] + a1_ref[0]
        s = s0_ref[0][:, 0:1] + s1_ref[0][:, 0:1]
        h_ref[...] = o / (s + 1e-16) + b_ref[...]

    return pl.pallas_call(
        body,
        grid=(NP // 512,),
        in_specs=[
            pl.BlockSpec((1, 512, H), lambda i: (0, i, 0)),
            pl.BlockSpec((1, 512, H), lambda i: (1, i, 0)),
            pl.BlockSpec((1, 512, 16), lambda i: (0, i, 0)),
            pl.BlockSpec((1, 512, 16), lambda i: (1, i, 0)),
            pl.BlockSpec((H,), lambda i: (0,)),
        ],
        out_specs=pl.BlockSpec((512, H), lambda i: (i, 0)),
        out_shape=jax.ShapeDtypeStruct((NP, H), jnp.float32),
    )(ao, ao, asum, asum, b)


def _sc_edge_kernel(xl_hbm, idx_hbm, xr_hbm, att_hbm,
                    out_hbm, s_hbm,
                    out_shared, s_shared, att_v,
                    xl0, xr0, es0, xl1, xr1, es1,
                    q0, q1, q2, q3,
                    sg0, sg1, ss0, ss1, si0, si1, si2, si3):
    cid = lax.axis_index("c")
    sid = lax.axis_index("s")

    pltpu.sync_copy(att_hbm, att_v)
    atts = [att_v[pl.ds(16 * j, 16)] for j in range(8)]

    def zero_row(i, _):
        for k8 in range(8):
            xl0[i, pl.ds(16 * k8, 16)] = jnp.zeros((16,), jnp.float32)
        es0[i, :] = jnp.zeros((16,), jnp.float32)
        return 0

    lax.fori_loop(0, C, zero_row, 0)
    r0 = sid * ROWS_PER_TILE
    for j in range(ROWS_PER_TILE // C):
        pltpu.sync_copy(xl0, out_shared.at[pl.ds(r0 + j * C, C)])
        pltpu.sync_copy(es0, s_shared.at[pl.ds(r0 + j * C, C)])
    plsc.subcore_barrier()

    wid = cid * 16 + sid
    gbase = wid * K

    bufs = ((xl0, xr0, es0, sg0, ss0), (xl1, xr1, es1, sg1, ss1))
    qs = ((q0, si0), (q1, si1), (q2, si2), (q3, si3))

    def issue_idx(c, qi):
        q, si = qs[qi]
        pltpu.async_copy(idx_hbm.at[gbase + c], q, si)

    def wait_idx(qi):
        q, si = qs[qi]
        pltpu.make_async_copy(idx_hbm.at[0], q, si).wait()

    def issue_gather(bi, qi):
        xl_b, xr_b, _, sg, _ = bufs[bi]
        q, _ = qs[qi]
        pltpu.async_copy(xl_hbm.at[q.at[0]], xl_b, sg)
        pltpu.async_copy(xr_hbm.at[q.at[1]], xr_b, sg)

    def wait_gather(bi, qi):
        xl_b, xr_b, _, sg, _ = bufs[bi]
        q, _ = qs[qi]
        pltpu.make_async_copy(xl_hbm.at[q.at[0]], xl_b, sg).wait()
        pltpu.make_async_copy(xr_hbm.at[q.at[1]], xr_b, sg).wait()

    def issue_scatter(bi, qi):
        xl_b, _, es_b, _, ss = bufs[bi]
        q, _ = qs[qi]
        pltpu.async_copy(xl_b, out_shared.at[q.at[1]], ss, add=True)
        pltpu.async_copy(es_b, s_shared.at[q.at[1]], ss, add=True)

    def wait_scatter(bi, qi):
        xl_b, _, es_b, _, ss = bufs[bi]
        q, _ = qs[qi]
        pltpu.make_async_copy(xl_b, out_shared.at[q.at[1]], ss).wait()
        pltpu.make_async_copy(es_b, s_shared.at[q.at[1]], ss).wait()

    def compute(bi):
        xl_b, xr_b, es_b, _, _ = bufs[bi]

        @plsc.parallel_loop(0, C, 1, unroll=2)
        def edge_body(i):
            als = [xl_b[i, pl.ds(16 * k8, 16)] for k8 in range(8)]
            p = jnp.zeros((16,), jnp.float32)
            for k8 in range(8):
                m = als[k8] + xr_b[i, pl.ds(16 * k8, 16)]
                m = jnp.maximum(m, 0.2 * m)
                p = p + m * atts[k8]
            logit = jnp.sum(p)
            ev = jnp.exp(lax.broadcast(logit, (16,)))
            for k8 in range(8):
                xl_b[i, pl.ds(16 * k8, 16)] = als[k8] * ev
            es_b[i, :] = ev

    issue_idx(0, 0)
    issue_idx(1, 1)
    issue_idx(2, 2)
    wait_idx(0)
    issue_gather(0, 0)

    T = K // 4

    def body(u, _):
        c0 = 4 * u

        wait_idx(1)

        @pl.when(u > 0)
        def _():
            wait_scatter(1, 3)
        issue_gather(1, 1)
        issue_idx(c0 + 3, 3)
        wait_gather(0, 0)
        compute(0)
        issue_scatter(0, 0)

        wait_idx(2)
        wait_scatter(0, 0)
        issue_gather(0, 2)

        @pl.when(c0 + 4 < K)
        def _():
            issue_idx(c0 + 4, 0)
        wait_gather(1, 1)
        compute(1)
        issue_scatter(1, 1)

        wait_idx(3)
        wait_scatter(1, 1)
        issue_gather(1, 3)

        @pl.when(c0 + 5 < K)
        def _():
            issue_idx(c0 + 5, 1)
        wait_gather(0, 2)
        compute(0)
        issue_scatter(0, 2)

        wait_scatter(0, 2)

        @pl.when(u + 1 < T)
        def _():
            wait_idx(0)
            issue_gather(0, 0)
            issue_idx(c0 + 6, 2)
        wait_gather(1, 3)
        compute(1)
        issue_scatter(1, 3)
        return 0

    lax.fori_loop(0, T, body, 0)
    wait_scatter(1, 3)

    plsc.subcore_barrier()
    pltpu.sync_copy(out_shared.at[pl.ds(r0, ROWS_PER_TILE)],
                    out_hbm.at[cid, pl.ds(r0, ROWS_PER_TILE)])
    pltpu.sync_copy(s_shared.at[pl.ds(r0, ROWS_PER_TILE)],
                    s_hbm.at[cid, pl.ds(r0, ROWS_PER_TILE)])


def _sc_edge(xl, xr, att, idx_pk):
    mesh = plsc.VectorSubcoreMesh(core_axis_name="c", subcore_axis_name="s")
    f = pl.kernel(
        _sc_edge_kernel,
        out_type=[
            jax.ShapeDtypeStruct((2, NP, H), jnp.float32),
            jax.ShapeDtypeStruct((2, NP, 16), jnp.float32),
        ],
        mesh=mesh,
        scratch_types=[
            pltpu.VMEM_SHARED((NP, H), jnp.float32),
            pltpu.VMEM_SHARED((NP, 16), jnp.float32),
            pltpu.VMEM((H,), jnp.float32),
            pltpu.VMEM((C, H), jnp.float32),
            pltpu.VMEM((C, H), jnp.float32),
            pltpu.VMEM((C, 16), jnp.float32),
            pltpu.VMEM((C, H), jnp.float32),
            pltpu.VMEM((C, H), jnp.float32),
            pltpu.VMEM((C, 16), jnp.float32),
            pltpu.VMEM((2, C), jnp.int32),
            pltpu.VMEM((2, C), jnp.int32),
            pltpu.VMEM((2, C), jnp.int32),
            pltpu.VMEM((2, C), jnp.int32),
            pltpu.SemaphoreType.DMA,
            pltpu.SemaphoreType.DMA,
            pltpu.SemaphoreType.DMA,
            pltpu.SemaphoreType.DMA,
            pltpu.SemaphoreType.DMA,
            pltpu.SemaphoreType.DMA,
            pltpu.SemaphoreType.DMA,
            pltpu.SemaphoreType.DMA,
        ],
        compiler_params=pltpu.CompilerParams(needs_layout_passes=False,
                                             use_tc_tiling_on_sc=False),
    )
    return f(xl, idx_pk, xr, att)


def kernel(x, edge_index, Wl1, Wr1, att1, b1, Wl2, Wr2, att2, b2,
           Wl3, Wr3, att3, b3):
    x = x.astype(jnp.float32)
    feat = jnp.zeros((NP, 8), jnp.float32).at[:N, :6].set(x[:, 4:10])
    wl1 = jnp.zeros((8, H), jnp.float32).at[:6].set(Wl1)
    wr1 = jnp.zeros((8, H), jnp.float32).at[:6].set(Wr1)

    loops = jnp.arange(N, dtype=jnp.int32)
    pad = EP - E_TOT
    src = jnp.concatenate([edge_index[0].astype(jnp.int32), loops,
                           jnp.zeros((pad,), jnp.int32)])
    dst = jnp.concatenate([edge_index[1].astype(jnp.int32), loops,
                           jnp.full((pad,), NP - 1, jnp.int32)])
    idx_pk = jnp.stack([src.reshape(NW * K, C), dst.reshape(NW * K, C)],
                       axis=1)

    xl, xr = _tc_matmul2(feat, wl1, wr1)
    ao, asum = _sc_edge(xl, xr, att1, idx_pk)
    xl, xr = _tc_finish_mid(ao, asum, b1, Wl2, Wr2)
    ao, asum = _sc_edge(xl, xr, att2, idx_pk)
    xl, xr = _tc_finish_mid(ao, asum, b2, Wl3, Wr3)
    ao, asum = _sc_edge(xl, xr, att3, idx_pk)
    h = _tc_finish_last(ao, asum, b3)
    return h[:N]


# speedup vs baseline: 23.8882x; 1.9350x over previous
import functools

import jax
import jax.numpy as jnp
from jax import lax
from jax.experimental import pallas as pl
from jax.experimental.pallas import tpu as pltpu
from jax.experimental.pallas import tpu_sc as plsc

N = 10000
NP = 10240
H = 128
C = 64
NW = 32
E_TOT = 320000 + N
K = 4 * (-(-E_TOT // (NW * C * 4)))
EP = NW * C * K
ROWS_PER_TILE = NP // 16


def _tc_matmul2(h, wl, wr):
    kd = h.shape[1]

    def body(h_ref, wl_ref, wr_ref, xl_ref, xr_ref):
        hb = h_ref[...]
        xl_ref[...] = jnp.dot(hb, wl_ref[...], preferred_element_type=jnp.float32)
        xr_ref[...] = jnp.dot(hb, wr_ref[...], preferred_element_type=jnp.float32)

    return pl.pallas_call(
        body,
        grid=(NP // 512,),
        in_specs=[
            pl.BlockSpec((512, kd), lambda i: (i, 0)),
            pl.BlockSpec((kd, H), lambda i: (0, 0)),
            pl.BlockSpec((kd, H), lambda i: (0, 0)),
        ],
        out_specs=[
            pl.BlockSpec((512, H), lambda i: (i, 0)),
            pl.BlockSpec((512, H), lambda i: (i, 0)),
        ],
        out_shape=[
            jax.ShapeDtypeStruct((NP, H), jnp.float32),
            jax.ShapeDtypeStruct((NP, H), jnp.float32),
        ],
    )(h, wl, wr)


def _tc_finish_mid(ao, asum, b, wl, wr):
    def body(a0_ref, a1_ref, s0_ref, s1_ref, b_ref, wl_ref, wr_ref,
             xl_ref, xr_ref):
        o = a0_ref[0] + a1_ref[0]
        s = s0_ref[0][:, 0:1] + s1_ref[0][:, 0:1]
        h = o / (s + 1e-16) + b_ref[...]
        h = jnp.maximum(h, 0.0)
        xl_ref[...] = jnp.dot(h, wl_ref[...], preferred_element_type=jnp.float32)
        xr_ref[...] = jnp.dot(h, wr_ref[...], preferred_element_type=jnp.float32)

    return pl.pallas_call(
        body,
        grid=(NP // 512,),
        in_specs=[
            pl.BlockSpec((1, 512, H), lambda i: (0, i, 0)),
            pl.BlockSpec((1, 512, H), lambda i: (1, i, 0)),
            pl.BlockSpec((1, 512, 16), lambda i: (0, i, 0)),
            pl.BlockSpec((1, 512, 16), lambda i: (1, i, 0)),
            pl.BlockSpec((H,), lambda i: (0,)),
            pl.BlockSpec((H, H), lambda i: (0, 0)),
            pl.BlockSpec((H, H), lambda i: (0, 0)),
        ],
        out_specs=[
            pl.BlockSpec((512, H), lambda i: (i, 0)),
            pl.BlockSpec((512, H), lambda i: (i, 0)),
        ],
        out_shape=[
            jax.ShapeDtypeStruct((NP, H), jnp.float32),
            jax.ShapeDtypeStruct((NP, H), jnp.float32),
        ],
    )(ao, ao, asum, asum, b, wl, wr)


def _tc_finish_last(ao, asum, b):
    def body(a0_ref, a1_ref, s0_ref, s1_ref, b_ref, h_ref):
        o = a0_ref[0] + a1_ref[0]
        s = s0_ref[0][:, 0:1] + s1_ref[0][:, 0:1]
        h_ref[...] = o / (s + 1e-16) + b_ref[...]

    return pl.pallas_call(
        body,
        grid=(NP // 512,),
        in_specs=[
            pl.BlockSpec((1, 512, H), lambda i: (0, i, 0)),
            pl.BlockSpec((1, 512, H), lambda i: (1, i, 0)),
            pl.BlockSpec((1, 512, 16), lambda i: (0, i, 0)),
            pl.BlockSpec((1, 512, 16), lambda i: (1, i, 0)),
            pl.BlockSpec((H,), lambda i: (0,)),
        ],
        out_specs=pl.BlockSpec((512, H), lambda i: (i, 0)),
        out_shape=jax.ShapeDtypeStruct((NP, H), jnp.float32),
    )(ao, ao, asum, asum, b)


def _sc_edge_kernel(xl_hbm, idx_hbm, xr_hbm, att_hbm,
                    out_hbm, s_hbm,
                    out_shared, s_shared, att_v,
                    xl0, xr0, es0, xl1, xr1, es1,
                    q0, q1, q2, q3,
                    sg0, sg1, ss0, ss1, si0, si1, si2, si3):
    cid = lax.axis_index("c")
    sid = lax.axis_index("s")

    pltpu.sync_copy(att_hbm, att_v)
    atts = [att_v[pl.ds(16 * j, 16)] for j in range(8)]

    def zero_row(i, _):
        for k8 in range(8):
            xl0[i, pl.ds(16 * k8, 16)] = jnp.zeros((16,), jnp.float32)
        es0[i, :] = jnp.zeros((16,), jnp.float32)
        return 0

    lax.fori_loop(0, C, zero_row, 0)
    r0 = sid * ROWS_PER_TILE
    for j in range(ROWS_PER_TILE // C):
        pltpu.sync_copy(xl0, out_shared.at[pl.ds(r0 + j * C, C)])
        pltpu.sync_copy(es0, s_shared.at[pl.ds(r0 + j * C, C)])
    plsc.subcore_barrier()

    wid = cid * 16 + sid
    gbase = wid * K

    bufs = ((xl0, xr0, es0, sg0, ss0), (xl1, xr1, es1, sg1, ss1))
    qs = ((q0, si0), (q1, si1), (q2, si2), (q3, si3))

    def issue_idx(c, qi):
        q, si = qs[qi]
        pltpu.async_copy(idx_hbm.at[gbase + c], q, si)

    def wait_idx(qi):
        q, si = qs[qi]
        pltpu.make_async_copy(idx_hbm.at[0], q, si).wait()

    def issue_gather(bi, qi):
        xl_b, xr_b, _, sg, _ = bufs[bi]
        q, _ = qs[qi]
        pltpu.async_copy(xl_hbm.at[q.at[0]], xl_b, sg)
        pltpu.async_copy(xr_hbm.at[q.at[1]], xr_b, sg)

    def wait_gather(bi, qi):
        xl_b, xr_b, _, sg, _ = bufs[bi]
        q, _ = qs[qi]
        pltpu.make_async_copy(xl_hbm.at[q.at[0]], xl_b, sg).wait()
        pltpu.make_async_copy(xr_hbm.at[q.at[1]], xr_b, sg).wait()

    def issue_scatter(bi, qi):
        xl_b, _, es_b, _, ss = bufs[bi]
        q, _ = qs[qi]
        pltpu.async_copy(xl_b, out_shared.at[q.at[1]], ss, add=True)
        pltpu.async_copy(es_b, s_shared.at[q.at[1]], ss, add=True)

    def wait_scatter(bi, qi):
        xl_b, _, es_b, _, ss = bufs[bi]
        q, _ = qs[qi]
        pltpu.make_async_copy(xl_b, out_shared.at[q.at[1]], ss).wait()
        pltpu.make_async_copy(es_b, s_shared.at[q.at[1]], ss).wait()

    def compute(bi):
        xl_b, xr_b, es_b, _, _ = bufs[bi]

        @plsc.parallel_loop(0, C, 1, unroll=2)
        def edge_body(i):
            als = [xl_b[i, pl.ds(16 * k8, 16)] for k8 in range(8)]
            p = jnp.zeros((16,), jnp.float32)
            for k8 in range(8):
                m = als[k8] + xr_b[i, pl.ds(16 * k8, 16)]
                m = jnp.maximum(m, 0.2 * m)
                p = p + m * atts[k8]
            logit = jnp.sum(p)
            ev = jnp.exp(lax.broadcast(logit, (16,)))
            for k8 in range(8):
                xl_b[i, pl.ds(16 * k8, 16)] = als[k8] * ev
            es_b[i, :] = ev

    issue_idx(0, 0)
    issue_idx(1, 1)
    issue_idx(2, 2)
    wait_idx(0)
    issue_gather(0, 0)

    T = K // 4

    def body(u, _):
        c0 = 4 * u

        wait_idx(1)

        @pl.when(u > 0)
        def _():
            wait_scatter(1, 3)
        issue_gather(1, 1)
        issue_idx(c0 + 3, 3)
        wait_gather(0, 0)
        compute(0)
        issue_scatter(0, 0)

        wait_idx(2)
        wait_scatter(0, 0)
        issue_gather(0, 2)

        @pl.when(c0 + 4 < K)
        def _():
            issue_idx(c0 + 4, 0)
        wait_gather(1, 1)
        compute(1)
        issue_scatter(1, 1)

        wait_idx(3)
        wait_scatter(1, 1)
        issue_gather(1, 3)

        @pl.when(c0 + 5 < K)
        def _():
            issue_idx(c0 + 5, 1)
        wait_gather(0, 2)
        compute(0)
        issue_scatter(0, 2)

        wait_scatter(0, 2)

        @pl.when(u + 1 < T)
        def _():
            wait_idx(0)
            issue_gather(0, 0)
            issue_idx(c0 + 6, 2)
        wait_gather(1, 3)
        compute(1)
        issue_scatter(1, 3)
        return 0

    lax.fori_loop(0, T, body, 0)
    wait_scatter(1, 3)

    plsc.subcore_barrier()
    pltpu.sync_copy(out_shared.at[pl.ds(r0, ROWS_PER_TILE)],
                    out_hbm.at[cid, pl.ds(r0, ROWS_PER_TILE)])
    pltpu.sync_copy(s_shared.at[pl.ds(r0, ROWS_PER_TILE)],
                    s_hbm.at[cid, pl.ds(r0, ROWS_PER_TILE)])


def _sc_edge(xl, xr, att, idx_pk):
    mesh = plsc.VectorSubcoreMesh(core_axis_name="c", subcore_axis_name="s")
    f = pl.kernel(
        _sc_edge_kernel,
        out_type=[
            jax.ShapeDtypeStruct((2, NP, H), jnp.float32),
            jax.ShapeDtypeStruct((2, NP, 16), jnp.float32),
        ],
        mesh=mesh,
        scratch_types=[
            pltpu.VMEM_SHARED((NP, H), jnp.float32),
            pltpu.VMEM_SHARED((NP, 16), jnp.float32),
            pltpu.VMEM((H,), jnp.float32),
            pltpu.VMEM((C, H), jnp.float32),
            pltpu.VMEM((C, H), jnp.float32),
            pltpu.VMEM((C, 16), jnp.float32),
            pltpu.VMEM((C, H), jnp.float32),
            pltpu.VMEM((C, H), jnp.float32),
            pltpu.VMEM((C, 16), jnp.float32),
            pltpu.VMEM((2, C), jnp.int32),
            pltpu.VMEM((2, C), jnp.int32),
            pltpu.VMEM((2, C), jnp.int32),
            pltpu.VMEM((2, C), jnp.int32),
            pltpu.SemaphoreType.DMA,
            pltpu.SemaphoreType.DMA,
            pltpu.SemaphoreType.DMA,
            pltpu.SemaphoreType.DMA,
            pltpu.SemaphoreType.DMA,
            pltpu.SemaphoreType.DMA,
            pltpu.SemaphoreType.DMA,
            pltpu.SemaphoreType.DMA,
        ],
        compiler_params=pltpu.CompilerParams(needs_layout_passes=False,
                                             use_tc_tiling_on_sc=False),
    )
    return f(xl, idx_pk, xr, att)


def kernel(x, edge_index, Wl1, Wr1, att1, b1, Wl2, Wr2, att2, b2,
           Wl3, Wr3, att3, b3):
    x = x.astype(jnp.float32)
    feat = jnp.zeros((NP, 8), jnp.float32).at[:N, :6].set(x[:, 4:10])
    wl1 = jnp.zeros((8, H), jnp.float32).at[:6].set(Wl1)
    wr1 = jnp.zeros((8, H), jnp.float32).at[:6].set(Wr1)

    loops = jnp.arange(N, dtype=jnp.int32)
    pad = EP - E_TOT
    pad_rows = N + jnp.arange(pad, dtype=jnp.int32) % (NP - N)
    src = jnp.concatenate([edge_index[0].astype(jnp.int32), loops, pad_rows])
    dst = jnp.concatenate([edge_index[1].astype(jnp.int32), loops, pad_rows])
    idx_pk = jnp.stack([src.reshape(NW * K, C), dst.reshape(NW * K, C)],
                       axis=1)

    xl, xr = _tc_matmul2(feat, wl1, wr1)
    ao, asum = _sc_edge(xl, xr, att1, idx_pk)
    xl, xr = _tc_finish_mid(ao, asum, b1, Wl2, Wr2)
    ao, asum = _sc_edge(xl, xr, att2, idx_pk)
    xl, xr = _tc_finish_mid(ao, asum, b2, Wl3, Wr3)
    ao, asum = _sc_edge(xl, xr, att3, idx_pk)
    h = _tc_finish_last(ao, asum, b3)
    return h[:N]
